# baseline (speedup 1.0000x reference)
import jax
import jax.numpy as jnp
from jax.experimental import pallas as pl
from jax.experimental.pallas import tpu as pltpu

_E = 16
_D = 2048
_TM = 2048


def _router_block(x_ref, wt_ref, b_ref, onehot_ref, plog_ref):
    x = x_ref[...]
    logits = jnp.dot(x, wt_ref[...], preferred_element_type=jnp.float32)
    logits = logits + b_ref[...]
    m = jnp.max(logits, axis=-1, keepdims=True)
    e = jnp.exp(logits - m)
    s = jnp.sum(e, axis=-1, keepdims=True)
    p = e / s
    pmax = jnp.max(p, axis=-1, keepdims=True)
    ii = jax.lax.broadcasted_iota(jnp.int32, p.shape, 1)
    idx = jnp.min(jnp.where(p == pmax, ii, _E), axis=-1, keepdims=True)
    onehot_ref[...] = (ii == idx).astype(jnp.int32)
    plog_ref[...] = pmax


def kernel(hidden_states, W, b):
    B, S, D = hidden_states.shape
    M = B * S
    x = hidden_states.reshape(M, D)
    wt = W.T
    b2 = b.reshape(1, _E)
    grid = (M // _TM,)
    onehot, plog = pl.pallas_call(
        _router_block,
        grid=grid,
        in_specs=[
            pl.BlockSpec((_TM, D), lambda i: (i, 0)),
            pl.BlockSpec((D, _E), lambda i: (0, 0)),
            pl.BlockSpec((1, _E), lambda i: (0, 0)),
        ],
        out_specs=[
            pl.BlockSpec((_TM, _E), lambda i: (i, 0)),
            pl.BlockSpec((_TM, 1), lambda i: (i, 0)),
        ],
        out_shape=[
            jax.ShapeDtypeStruct((M, _E), jnp.int32),
            jax.ShapeDtypeStruct((M, 1), jnp.float32),
        ],
        compiler_params=pltpu.CompilerParams(
            dimension_semantics=("parallel",),
        ),
    )(x, wt, b2)
    return (onehot.reshape(B, S, 1, _E), plog.reshape(B, S, 1))
